# Initial kernel scaffold; baseline (speedup 1.0000x reference)
import jax
import jax.numpy as jnp
from jax.experimental import pallas as pl


def kernel(x, edge_index, W1, b1, W2, b2):
    raise NotImplementedError("write your pallas kernel here")


# speedup vs baseline: 26.2970x; 26.2970x over previous
import functools
import jax
import jax.numpy as jnp
from jax import lax
from jax.experimental import pallas as pl
from jax.experimental.pallas import tpu as pltpu
from jax.experimental.pallas import tpu_sc as plsc

N = 100000
E = 6400000
IN_DIM = 4
HID = 32

NC = 2
NS = 16
LANES = 128
SUB = 8
CHUNK = SUB * LANES

EP = ((E + NC * NS * CHUNK - 1) // (NC * NS * CHUNK)) * (NC * NS * CHUNK)
IDXROWS = EP // LANES
NACC = ((N + 1) + NS * 8 - 1) // (NS * 8) * (NS * 8)
ZCH = NACC // NS

_MESH = plsc.VectorSubcoreMesh(core_axis_name="c", subcore_axis_name="s")


def _sc_agg(table, idx2d, dst2d, zeros, split_edges: bool, rows_per_sub: int):
    n_chunks = rows_per_sub // SUB

    @functools.partial(
        pl.kernel,
        out_type=jax.ShapeDtypeStruct((NC, NACC, 16), jnp.float32),
        mesh=_MESH,
        scratch_types=[
            pltpu.VMEM((SUB, LANES), jnp.int32),
            pltpu.VMEM((SUB, LANES), jnp.int32),
            pltpu.VMEM((CHUNK, 16), jnp.float32),
            pltpu.VMEM_SHARED((NACC, 16), jnp.float32),
            pltpu.SemaphoreType.DMA,
        ],
        compiler_params=pltpu.CompilerParams(use_tc_tiling_on_sc=False),
    )
    def k(table_h, idx_h, dst_h, zeros_h, out_h, sbuf, dbuf, rows_v, acc, sem):
        c = lax.axis_index("c")
        s = lax.axis_index("s")
        zb = s * ZCH
        pltpu.sync_copy(zeros_h.at[pl.ds(zb, ZCH)], acc.at[pl.ds(zb, ZCH)])
        plsc.subcore_barrier()

        if split_edges:
            row0 = (c * NS + s) * rows_per_sub
        else:
            row0 = s * rows_per_sub

        def chunk(g, carry):
            r0 = row0 + g * SUB
            pltpu.sync_copy(idx_h.at[c, pl.ds(r0, SUB)], sbuf)
            pltpu.sync_copy(dst_h.at[pl.ds(r0, SUB)], dbuf)
            cps = [
                pltpu.async_copy(
                    table_h.at[sbuf.at[j]],
                    rows_v.at[pl.ds(j * LANES, LANES)],
                    sem,
                )
                for j in range(SUB)
            ]
            for cp in cps:
                cp.wait()
            for j in range(SUB):
                pltpu.sync_copy(
                    rows_v.at[pl.ds(j * LANES, LANES)],
                    acc.at[dbuf.at[j]],
                    add=True,
                )
            return carry

        lax.fori_loop(0, n_chunks, chunk, 0)
        plsc.subcore_barrier()
        pltpu.sync_copy(acc.at[pl.ds(zb, ZCH)], out_h.at[c, pl.ds(zb, ZCH)])

    return k(table, idx2d, dst2d, zeros)


BR = 1000


def _tc_layer1(P, W1p, b1):
    def body(p_ref, w_ref, b_ref, h_ref, dinv_ref):
        sblk = p_ref[0] + p_ref[1]
        dinv = 1.0 / jnp.maximum(sblk[:, 4:5], 1.0)
        mean = sblk * dinv
        h = jnp.dot(mean, w_ref[...], preferred_element_type=jnp.float32)
        h_ref[...] = jnp.maximum(h + b_ref[...], 0.0)
        dinv_ref[...] = jnp.broadcast_to(dinv, (BR, 8))

    return pl.pallas_call(
        body,
        grid=(N // BR,),
        in_specs=[
            pl.BlockSpec((NC, BR, 16), lambda i: (0, i, 0)),
            pl.BlockSpec((16, HID), lambda i: (0, 0)),
            pl.BlockSpec((1, HID), lambda i: (0, 0)),
        ],
        out_specs=[
            pl.BlockSpec((BR, HID), lambda i: (i, 0)),
            pl.BlockSpec((BR, 8), lambda i: (i, 0)),
        ],
        out_shape=[
            jax.ShapeDtypeStruct((N, HID), jnp.float32),
            jax.ShapeDtypeStruct((N, 8), jnp.float32),
        ],
    )(P, W1p, b1)


def _tc_layer2(A, dinv8, W2s, b2):
    def body(a_ref, d_ref, w_ref, b_ref, o_ref):
        acc = jnp.dot(a_ref[0], w_ref[0], preferred_element_type=jnp.float32)
        acc += jnp.dot(a_ref[1], w_ref[1], preferred_element_type=jnp.float32)
        o_ref[...] = jnp.maximum(acc * d_ref[:, 0:1] + b_ref[...], 0.0)

    return pl.pallas_call(
        body,
        grid=(N // BR,),
        in_specs=[
            pl.BlockSpec((NC, BR, 16), lambda i: (0, i, 0)),
            pl.BlockSpec((BR, 8), lambda i: (i, 0)),
            pl.BlockSpec((NC, 16, HID), lambda i: (0, 0, 0)),
            pl.BlockSpec((1, HID), lambda i: (0, 0)),
        ],
        out_specs=pl.BlockSpec((BR, HID), lambda i: (i, 0)),
        out_shape=jax.ShapeDtypeStruct((N, HID), jnp.float32),
    )(A, dinv8, W2s, b2)


@jax.jit
def kernel(x, edge_index, W1, b1, W2, b2):
    src = edge_index[0]
    dst = edge_index[1]

    pad = EP - E
    src_p = jnp.concatenate([src, jnp.zeros((pad,), jnp.int32)])
    dst_p = jnp.concatenate([dst, jnp.full((pad,), N, jnp.int32)])
    dst2d = dst_p.reshape(IDXROWS, LANES)

    src1 = jnp.broadcast_to(src_p.reshape(1, IDXROWS, LANES), (NC, IDXROWS, LANES))
    src2 = (2 * src_p)[None, :] + jnp.arange(NC, dtype=jnp.int32)[:, None]
    src2 = src2.reshape(NC, IDXROWS, LANES)

    zeros = jnp.zeros((NACC, 16), jnp.float32)

    xpad = jnp.pad(x, ((0, 0), (0, 16 - IN_DIM))).at[:, 4].set(1.0)

    P = _sc_agg(xpad, src1, dst2d, zeros, split_edges=True,
                rows_per_sub=IDXROWS // (NC * NS))

    W1p = jnp.pad(W1, ((0, 16 - IN_DIM), (0, 0)))
    h1, dinv8 = _tc_layer1(P, W1p, b1.reshape(1, HID))

    h1tab = h1.reshape(2 * N, 16)
    A = _sc_agg(h1tab, src2, dst2d, zeros, split_edges=False,
                rows_per_sub=IDXROWS // NS)

    W2s = W2.reshape(NC, 16, HID)
    return _tc_layer2(A, dinv8, W2s, b2.reshape(1, HID))


# speedup vs baseline: 32.2967x; 1.2281x over previous
import functools
import jax
import jax.numpy as jnp
from jax import lax
from jax.experimental import pallas as pl
from jax.experimental.pallas import tpu as pltpu
from jax.experimental.pallas import tpu_sc as plsc

N = 100000
E = 6400000
IN_DIM = 4
HID = 32

NC = 2
NS = 16
LANES = 128
SUB = 6
CHUNK = SUB * LANES

BODY = 2 * CHUNK
EP = ((E + NC * NS * BODY - 1) // (NC * NS * BODY)) * (NC * NS * BODY)
IDXROWS = EP // LANES
NACC = ((N + 1) + NS * 8 - 1) // (NS * 8) * (NS * 8)
ZCH = NACC // NS

_MESH = plsc.VectorSubcoreMesh(core_axis_name="c", subcore_axis_name="s")


def _sc_agg(table, idx2d, dst2d, zeros, split_edges: bool, rows_per_sub: int):
    n_bodies = rows_per_sub // (2 * SUB)
    stacked = table.shape[0] == NC

    @functools.partial(
        pl.kernel,
        out_type=jax.ShapeDtypeStruct((NC, NACC, 16), jnp.float32),
        mesh=_MESH,
        scratch_types=[
            pltpu.VMEM((2 * SUB, LANES), jnp.int32),
            pltpu.VMEM((2 * SUB, LANES), jnp.int32),
            pltpu.VMEM((BODY, 16), jnp.float32),
            pltpu.VMEM_SHARED((NACC, 16), jnp.float32),
            pltpu.SemaphoreType.DMA,
            pltpu.SemaphoreType.DMA,
            pltpu.SemaphoreType.DMA,
            pltpu.SemaphoreType.DMA,
        ],
        compiler_params=pltpu.CompilerParams(use_tc_tiling_on_sc=False),
    )
    def k(table_h, idx_h, dst_h, zeros_h, out_h,
          sbuf, dbuf, rows_v, acc, gsem0, gsem1, ssem0, ssem1):
        c = lax.axis_index("c")
        s = lax.axis_index("s")
        tab = table_h.at[c] if stacked else table_h.at[0]
        zb = s * ZCH
        pltpu.sync_copy(zeros_h.at[pl.ds(zb, ZCH)], acc.at[pl.ds(zb, ZCH)])
        plsc.subcore_barrier()

        if split_edges:
            row0 = (c * NS + s) * rows_per_sub
        else:
            row0 = s * rows_per_sub

        def body(i, carry):
            r0 = row0 + i * 2 * SUB
            pltpu.sync_copy(idx_h.at[pl.ds(r0, 2 * SUB)], sbuf)
            pltpu.sync_copy(dst_h.at[pl.ds(r0, 2 * SUB)], dbuf)
            g0 = [
                pltpu.async_copy(tab.at[sbuf.at[j]],
                                 rows_v.at[pl.ds(j * LANES, LANES)], gsem0)
                for j in range(SUB)
            ]
            g1 = [
                pltpu.async_copy(tab.at[sbuf.at[SUB + j]],
                                 rows_v.at[pl.ds(CHUNK + j * LANES, LANES)], gsem1)
                for j in range(SUB)
            ]
            for cp in g0:
                cp.wait()
            s0 = [
                pltpu.async_copy(rows_v.at[pl.ds(j * LANES, LANES)],
                                 acc.at[dbuf.at[j]], ssem0, add=True)
                for j in range(SUB)
            ]
            for cp in g1:
                cp.wait()
            s1 = [
                pltpu.async_copy(rows_v.at[pl.ds(CHUNK + j * LANES, LANES)],
                                 acc.at[dbuf.at[SUB + j]], ssem1, add=True)
                for j in range(SUB)
            ]
            for cp in s0:
                cp.wait()
            for cp in s1:
                cp.wait()
            return carry

        lax.fori_loop(0, n_bodies, body, 0)
        plsc.subcore_barrier()
        pltpu.sync_copy(acc.at[pl.ds(zb, ZCH)], out_h.at[c, pl.ds(zb, ZCH)])

    return k(table, idx2d, dst2d, zeros)


BR = 1000


def _tc_layer1(P, W1p, b1):
    def body(p_ref, w_ref, b_ref, h_ref, dinv_ref):
        sblk = p_ref[0] + p_ref[1]
        dinv = 1.0 / jnp.maximum(sblk[:, 4:5], 1.0)
        mean = sblk * dinv
        h = jnp.dot(mean, w_ref[...], preferred_element_type=jnp.float32)
        h = jnp.maximum(h + b_ref[...], 0.0)
        h_ref[0] = h[:, :16]
        h_ref[1] = h[:, 16:]
        dinv_ref[...] = jnp.broadcast_to(dinv, (BR, 8))

    return pl.pallas_call(
        body,
        grid=(N // BR,),
        in_specs=[
            pl.BlockSpec((NC, BR, 16), lambda i: (0, i, 0)),
            pl.BlockSpec((16, HID), lambda i: (0, 0)),
            pl.BlockSpec((1, HID), lambda i: (0, 0)),
        ],
        out_specs=[
            pl.BlockSpec((NC, BR, 16), lambda i: (0, i, 0)),
            pl.BlockSpec((BR, 8), lambda i: (i, 0)),
        ],
        out_shape=[
            jax.ShapeDtypeStruct((NC, N, 16), jnp.float32),
            jax.ShapeDtypeStruct((N, 8), jnp.float32),
        ],
    )(P, W1p, b1)


def _tc_layer2(A, dinv8, W2s, b2):
    def body(a_ref, d_ref, w_ref, b_ref, o_ref):
        acc = jnp.dot(a_ref[0], w_ref[0], preferred_element_type=jnp.float32)
        acc += jnp.dot(a_ref[1], w_ref[1], preferred_element_type=jnp.float32)
        o_ref[...] = jnp.maximum(acc * d_ref[:, 0:1] + b_ref[...], 0.0)

    return pl.pallas_call(
        body,
        grid=(N // BR,),
        in_specs=[
            pl.BlockSpec((NC, BR, 16), lambda i: (0, i, 0)),
            pl.BlockSpec((BR, 8), lambda i: (i, 0)),
            pl.BlockSpec((NC, 16, HID), lambda i: (0, 0, 0)),
            pl.BlockSpec((1, HID), lambda i: (0, 0)),
        ],
        out_specs=pl.BlockSpec((BR, HID), lambda i: (i, 0)),
        out_shape=jax.ShapeDtypeStruct((N, HID), jnp.float32),
    )(A, dinv8, W2s, b2)


@jax.jit
def kernel(x, edge_index, W1, b1, W2, b2):
    src = edge_index[0]
    dst = edge_index[1]

    pad = EP - E
    src_p = jnp.concatenate([src, jnp.zeros((pad,), jnp.int32)])
    dst_p = jnp.concatenate([dst, jnp.full((pad,), N, jnp.int32)])
    src2d = src_p.reshape(IDXROWS, LANES)
    dst2d = dst_p.reshape(IDXROWS, LANES)

    zeros = jnp.zeros((NACC, 16), jnp.float32)

    xpad = jnp.pad(x, ((0, 0), (0, 16 - IN_DIM))).at[:, 4].set(1.0)

    P = _sc_agg(xpad.reshape(1, N, 16), src2d, dst2d, zeros, split_edges=True,
                rows_per_sub=IDXROWS // (NC * NS))

    W1p = jnp.pad(W1, ((0, 16 - IN_DIM), (0, 0)))
    h1t, dinv8 = _tc_layer1(P, W1p, b1.reshape(1, HID))

    A = _sc_agg(h1t, src2d, dst2d, zeros, split_edges=False,
                rows_per_sub=IDXROWS // NS)

    W2s = W2.reshape(NC, 16, HID)
    return _tc_layer2(A, dinv8, W2s, b2.reshape(1, HID))


# speedup vs baseline: 34.1686x; 1.0580x over previous
import functools
import jax
import jax.numpy as jnp
from jax import lax
from jax.experimental import pallas as pl
from jax.experimental.pallas import tpu as pltpu
from jax.experimental.pallas import tpu_sc as plsc

N = 100000
E = 6400000
IN_DIM = 4
HID = 32

NC = 2
NS = 16
LANES = 128
SUB = 6
CHUNK = SUB * LANES

BODY = 2 * CHUNK
EP = ((E + NC * NS * BODY - 1) // (NC * NS * BODY)) * (NC * NS * BODY)
IDXROWS = EP // LANES
NACC = ((N + 1) + NS * 8 - 1) // (NS * 8) * (NS * 8)
ZCH = NACC // NS

_MESH = plsc.VectorSubcoreMesh(core_axis_name="c", subcore_axis_name="s")


def _sc_agg(table, idx2d, dst2d, zeros, split_edges: bool, rows_per_sub: int):
    n_bodies = rows_per_sub // (2 * SUB)
    stacked = table.shape[0] == NC

    @functools.partial(
        pl.kernel,
        out_type=jax.ShapeDtypeStruct((NC, NACC, 16), jnp.float32),
        mesh=_MESH,
        scratch_types=[
            pltpu.VMEM((2 * SUB, LANES), jnp.int32),
            pltpu.VMEM((2 * SUB, LANES), jnp.int32),
            pltpu.VMEM((BODY, 16), jnp.float32),
            pltpu.VMEM_SHARED((NACC, 16), jnp.float32),
            pltpu.SemaphoreType.DMA,
            pltpu.SemaphoreType.DMA,
            pltpu.SemaphoreType.DMA,
            pltpu.SemaphoreType.DMA,
        ],
        compiler_params=pltpu.CompilerParams(use_tc_tiling_on_sc=False),
    )
    def k(table_h, idx_h, dst_h, zeros_h, out_h,
          sbuf, dbuf, rows_v, acc, gsem0, gsem1, ssem0, ssem1):
        c = lax.axis_index("c")
        s = lax.axis_index("s")
        tab = table_h.at[c] if stacked else table_h.at[0]
        zb = s * ZCH
        pltpu.sync_copy(zeros_h.at[pl.ds(zb, ZCH)], acc.at[pl.ds(zb, ZCH)])
        plsc.subcore_barrier()

        if split_edges:
            row0 = (c * NS + s) * rows_per_sub
        else:
            row0 = s * rows_per_sub

        pltpu.sync_copy(idx_h.at[pl.ds(row0, 2 * SUB)], sbuf)
        pltpu.sync_copy(dst_h.at[pl.ds(row0, 2 * SUB)], dbuf)

        def body(i, carry):
            g0 = [
                pltpu.async_copy(tab.at[sbuf.at[j]],
                                 rows_v.at[pl.ds(j * LANES, LANES)], gsem0)
                for j in range(SUB)
            ]
            g1 = [
                pltpu.async_copy(tab.at[sbuf.at[SUB + j]],
                                 rows_v.at[pl.ds(CHUNK + j * LANES, LANES)], gsem1)
                for j in range(SUB)
            ]
            for cp in g0:
                cp.wait()
            s0 = [
                pltpu.async_copy(rows_v.at[pl.ds(j * LANES, LANES)],
                                 acc.at[dbuf.at[j]], ssem0, add=True)
                for j in range(SUB)
            ]
            for cp in g1:
                cp.wait()
            s1 = [
                pltpu.async_copy(rows_v.at[pl.ds(CHUNK + j * LANES, LANES)],
                                 acc.at[dbuf.at[SUB + j]], ssem1, add=True)
                for j in range(SUB)
            ]
            r1 = row0 + (i + 1) * 2 * SUB
            pltpu.sync_copy(idx_h.at[pl.ds(r1, 2 * SUB)], sbuf)
            for cp in s0:
                cp.wait()
            for cp in s1:
                cp.wait()
            pltpu.sync_copy(dst_h.at[pl.ds(r1, 2 * SUB)], dbuf)
            return carry

        lax.fori_loop(0, n_bodies, body, 0)
        plsc.subcore_barrier()
        pltpu.sync_copy(acc.at[pl.ds(zb, ZCH)], out_h.at[c, pl.ds(zb, ZCH)])

    return k(table, idx2d, dst2d, zeros)


BR = 1000


def _tc_layer1(P, W1p, b1):
    def body(p_ref, w_ref, b_ref, h_ref, dinv_ref):
        sblk = p_ref[0] + p_ref[1]
        dinv = 1.0 / jnp.maximum(sblk[:, 4:5], 1.0)
        mean = sblk * dinv
        h = jnp.dot(mean, w_ref[...], preferred_element_type=jnp.float32)
        h = jnp.maximum(h + b_ref[...], 0.0)
        h_ref[0] = h[:, :16]
        h_ref[1] = h[:, 16:]
        dinv_ref[...] = jnp.broadcast_to(dinv, (BR, 8))

    return pl.pallas_call(
        body,
        grid=(N // BR,),
        in_specs=[
            pl.BlockSpec((NC, BR, 16), lambda i: (0, i, 0)),
            pl.BlockSpec((16, HID), lambda i: (0, 0)),
            pl.BlockSpec((1, HID), lambda i: (0, 0)),
        ],
        out_specs=[
            pl.BlockSpec((NC, BR, 16), lambda i: (0, i, 0)),
            pl.BlockSpec((BR, 8), lambda i: (i, 0)),
        ],
        out_shape=[
            jax.ShapeDtypeStruct((NC, N, 16), jnp.float32),
            jax.ShapeDtypeStruct((N, 8), jnp.float32),
        ],
    )(P, W1p, b1)


def _tc_layer2(A, dinv8, W2s, b2):
    def body(a_ref, d_ref, w_ref, b_ref, o_ref):
        acc = jnp.dot(a_ref[0], w_ref[0], preferred_element_type=jnp.float32)
        acc += jnp.dot(a_ref[1], w_ref[1], preferred_element_type=jnp.float32)
        o_ref[...] = jnp.maximum(acc * d_ref[:, 0:1] + b_ref[...], 0.0)

    return pl.pallas_call(
        body,
        grid=(N // BR,),
        in_specs=[
            pl.BlockSpec((NC, BR, 16), lambda i: (0, i, 0)),
            pl.BlockSpec((BR, 8), lambda i: (i, 0)),
            pl.BlockSpec((NC, 16, HID), lambda i: (0, 0, 0)),
            pl.BlockSpec((1, HID), lambda i: (0, 0)),
        ],
        out_specs=pl.BlockSpec((BR, HID), lambda i: (i, 0)),
        out_shape=jax.ShapeDtypeStruct((N, HID), jnp.float32),
    )(A, dinv8, W2s, b2)


@jax.jit
def kernel(x, edge_index, W1, b1, W2, b2):
    src = edge_index[0]
    dst = edge_index[1]

    pad = EP + BODY - E
    src_p = jnp.concatenate([src, jnp.zeros((pad,), jnp.int32)])
    dst_p = jnp.concatenate([dst, jnp.full((pad,), N, jnp.int32)])
    src2d = src_p.reshape(IDXROWS + 2 * SUB, LANES)
    dst2d = dst_p.reshape(IDXROWS + 2 * SUB, LANES)

    zeros = jnp.zeros((NACC, 16), jnp.float32)

    xpad = jnp.pad(x, ((0, 0), (0, 16 - IN_DIM))).at[:, 4].set(1.0)

    P = _sc_agg(xpad.reshape(1, N, 16), src2d, dst2d, zeros, split_edges=True,
                rows_per_sub=IDXROWS // (NC * NS))

    W1p = jnp.pad(W1, ((0, 16 - IN_DIM), (0, 0)))
    h1t, dinv8 = _tc_layer1(P, W1p, b1.reshape(1, HID))

    A = _sc_agg(h1t, src2d, dst2d, zeros, split_edges=False,
                rows_per_sub=IDXROWS // NS)

    W2s = W2.reshape(NC, 16, HID)
    return _tc_layer2(A, dinv8, W2s, b2.reshape(1, HID))


# speedup vs baseline: 34.8980x; 1.0213x over previous
import functools
import jax
import jax.numpy as jnp
from jax import lax
from jax.experimental import pallas as pl
from jax.experimental.pallas import tpu as pltpu
from jax.experimental.pallas import tpu_sc as plsc

N = 100000
E = 6400000
IN_DIM = 4
HID = 32

NC = 2
NS = 16
LANES = 128
SUB = 6
CHUNK = SUB * LANES

BODY = CHUNK
EP = ((E + NC * NS * BODY - 1) // (NC * NS * BODY)) * (NC * NS * BODY)
IDXROWS = EP // LANES
NACC = ((N + 1) + NS * 8 - 1) // (NS * 8) * (NS * 8)
ZCH = NACC // NS

_MESH = plsc.VectorSubcoreMesh(core_axis_name="c", subcore_axis_name="s")


def _sc_agg(table, idx2d, dst2d, zeros, split_edges: bool, rows_per_sub: int):
    n_bodies = rows_per_sub // SUB
    stacked = table.shape[0] == NC

    @functools.partial(
        pl.kernel,
        out_type=jax.ShapeDtypeStruct((NC, NACC, 16), jnp.float32),
        mesh=_MESH,
        scratch_types=[
            pltpu.VMEM((2, SUB, LANES), jnp.int32),
            pltpu.VMEM((2, SUB, LANES), jnp.int32),
            pltpu.VMEM((2 * CHUNK, 16), jnp.float32),
            pltpu.VMEM_SHARED((NACC, 16), jnp.float32),
            pltpu.SemaphoreType.DMA,
            pltpu.SemaphoreType.DMA,
        ],
        compiler_params=pltpu.CompilerParams(use_tc_tiling_on_sc=False),
    )
    def k(table_h, idx_h, dst_h, zeros_h, out_h,
          sbuf, dbuf, rows_v, acc, gsem, ssem):
        c = lax.axis_index("c")
        s = lax.axis_index("s")
        tab = table_h.at[c] if stacked else table_h.at[0]
        zb = s * ZCH
        pltpu.sync_copy(zeros_h.at[pl.ds(zb, ZCH)], acc.at[pl.ds(zb, ZCH)])
        plsc.subcore_barrier()

        if split_edges:
            row0 = (c * NS + s) * rows_per_sub
        else:
            row0 = s * rows_per_sub

        def drain_scatters(sem):
            for j in range(SUB):
                pltpu.make_async_copy(
                    zeros_h.at[pl.ds(0, LANES)],
                    rows_v.at[pl.ds(j * LANES, LANES)], sem).wait()

        pltpu.sync_copy(idx_h.at[pl.ds(row0, SUB)], sbuf.at[0])

        def body(i, carry):
            p = i & 1

            @pl.when(i > 1)
            def _():
                drain_scatters(ssem)

            g = [
                pltpu.async_copy(tab.at[sbuf.at[p, j]],
                                 rows_v.at[pl.ds(p * CHUNK + j * LANES, LANES)],
                                 gsem)
                for j in range(SUB)
            ]
            pltpu.sync_copy(dst_h.at[pl.ds(row0 + i * SUB, SUB)], dbuf.at[p])
            for cp in g:
                cp.wait()
            for j in range(SUB):
                pltpu.async_copy(rows_v.at[pl.ds(p * CHUNK + j * LANES, LANES)],
                                 acc.at[dbuf.at[p, j]], ssem, add=True)
            pltpu.sync_copy(idx_h.at[pl.ds(row0 + (i + 1) * SUB, SUB)],
                            sbuf.at[1 - p])
            return carry

        lax.fori_loop(0, n_bodies, body, 0)
        drain_scatters(ssem)
        drain_scatters(ssem)
        plsc.subcore_barrier()
        pltpu.sync_copy(acc.at[pl.ds(zb, ZCH)], out_h.at[c, pl.ds(zb, ZCH)])

    return k(table, idx2d, dst2d, zeros)


BR = 1000


def _tc_layer1(P, W1p, b1):
    def body(p_ref, w_ref, b_ref, h_ref, dinv_ref):
        sblk = p_ref[0] + p_ref[1]
        dinv = 1.0 / jnp.maximum(sblk[:, 4:5], 1.0)
        mean = sblk * dinv
        h = jnp.dot(mean, w_ref[...], preferred_element_type=jnp.float32)
        h = jnp.maximum(h + b_ref[...], 0.0)
        h_ref[0] = h[:, :16]
        h_ref[1] = h[:, 16:]
        dinv_ref[...] = jnp.broadcast_to(dinv, (BR, 8))

    return pl.pallas_call(
        body,
        grid=(N // BR,),
        in_specs=[
            pl.BlockSpec((NC, BR, 16), lambda i: (0, i, 0)),
            pl.BlockSpec((16, HID), lambda i: (0, 0)),
            pl.BlockSpec((1, HID), lambda i: (0, 0)),
        ],
        out_specs=[
            pl.BlockSpec((NC, BR, 16), lambda i: (0, i, 0)),
            pl.BlockSpec((BR, 8), lambda i: (i, 0)),
        ],
        out_shape=[
            jax.ShapeDtypeStruct((NC, N, 16), jnp.float32),
            jax.ShapeDtypeStruct((N, 8), jnp.float32),
        ],
    )(P, W1p, b1)


def _tc_layer2(A, dinv8, W2s, b2):
    def body(a_ref, d_ref, w_ref, b_ref, o_ref):
        acc = jnp.dot(a_ref[0], w_ref[0], preferred_element_type=jnp.float32)
        acc += jnp.dot(a_ref[1], w_ref[1], preferred_element_type=jnp.float32)
        o_ref[...] = jnp.maximum(acc * d_ref[:, 0:1] + b_ref[...], 0.0)

    return pl.pallas_call(
        body,
        grid=(N // BR,),
        in_specs=[
            pl.BlockSpec((NC, BR, 16), lambda i: (0, i, 0)),
            pl.BlockSpec((BR, 8), lambda i: (i, 0)),
            pl.BlockSpec((NC, 16, HID), lambda i: (0, 0, 0)),
            pl.BlockSpec((1, HID), lambda i: (0, 0)),
        ],
        out_specs=pl.BlockSpec((BR, HID), lambda i: (i, 0)),
        out_shape=jax.ShapeDtypeStruct((N, HID), jnp.float32),
    )(A, dinv8, W2s, b2)


@jax.jit
def kernel(x, edge_index, W1, b1, W2, b2):
    src = edge_index[0]
    dst = edge_index[1]

    pad = EP + BODY - E
    src_p = jnp.concatenate([src, jnp.zeros((pad,), jnp.int32)])
    dst_p = jnp.concatenate([dst, jnp.full((pad,), N, jnp.int32)])
    src2d = src_p.reshape(IDXROWS + SUB, LANES)
    dst2d = dst_p.reshape(IDXROWS + SUB, LANES)

    zeros = jnp.zeros((NACC, 16), jnp.float32)

    xpad = jnp.pad(x, ((0, 0), (0, 16 - IN_DIM))).at[:, 4].set(1.0)

    P = _sc_agg(xpad.reshape(1, N, 16), src2d, dst2d, zeros, split_edges=True,
                rows_per_sub=IDXROWS // (NC * NS))

    W1p = jnp.pad(W1, ((0, 16 - IN_DIM), (0, 0)))
    h1t, dinv8 = _tc_layer1(P, W1p, b1.reshape(1, HID))

    A = _sc_agg(h1t, src2d, dst2d, zeros, split_edges=False,
                rows_per_sub=IDXROWS // NS)

    W2s = W2.reshape(NC, 16, HID)
    return _tc_layer2(A, dinv8, W2s, b2.reshape(1, HID))


# speedup vs baseline: 40.3745x; 1.1569x over previous
import functools
import jax
import jax.numpy as jnp
from jax import lax
from jax.experimental import pallas as pl
from jax.experimental.pallas import tpu as pltpu
from jax.experimental.pallas import tpu_sc as plsc

N = 100000
E = 6400000
IN_DIM = 4
HID = 32

NC = 2
NS = 16
LANES = 128
SUB = 6
CHUNK = SUB * LANES

BODY = CHUNK
EP = ((E + NC * NS * BODY - 1) // (NC * NS * BODY)) * (NC * NS * BODY)
IDXROWS = EP // LANES
NACC = ((N + 1) + NS * 8 - 1) // (NS * 8) * (NS * 8)
ZCH = NACC // NS

_MESH = plsc.VectorSubcoreMesh(core_axis_name="c", subcore_axis_name="s")


def _sc_agg(table, idx2d, dst2d, zeros, split_edges: bool, rows_per_sub: int):
    n_bodies = rows_per_sub // SUB
    stacked = table.shape[0] == NC

    @functools.partial(
        pl.kernel,
        out_type=jax.ShapeDtypeStruct((NC, NACC, 16), jnp.float32),
        mesh=_MESH,
        scratch_types=[
            pltpu.VMEM((2, SUB, LANES), jnp.int32),
            pltpu.VMEM((2, SUB, LANES), jnp.int32),
            pltpu.VMEM((2 * CHUNK, 16), jnp.float32),
            pltpu.VMEM_SHARED((NACC, 16), jnp.float32),
            pltpu.SemaphoreType.DMA,
            pltpu.SemaphoreType.DMA,
            pltpu.SemaphoreType.DMA,
            pltpu.SemaphoreType.DMA,
        ],
        compiler_params=pltpu.CompilerParams(use_tc_tiling_on_sc=False),
    )
    def k(table_h, idx_h, dst_h, zeros_h, out_h,
          sbuf, dbuf, rows_v, acc, gsem, ssem, isem, dsem):
        c = lax.axis_index("c")
        s = lax.axis_index("s")
        tab = table_h.at[c] if stacked else table_h.at[0]
        zb = s * ZCH
        pltpu.sync_copy(zeros_h.at[pl.ds(zb, ZCH)], acc.at[pl.ds(zb, ZCH)])
        plsc.subcore_barrier()

        if split_edges:
            row0 = (c * NS + s) * rows_per_sub
        else:
            row0 = s * rows_per_sub

        def drain_scatters(sem):
            for j in range(SUB):
                pltpu.make_async_copy(
                    zeros_h.at[pl.ds(0, LANES)],
                    rows_v.at[pl.ds(j * LANES, LANES)], sem).wait()

        def drain_ibuf(buf, sem):
            pltpu.make_async_copy(idx_h.at[pl.ds(row0, SUB)], buf, sem).wait()

        pltpu.sync_copy(idx_h.at[pl.ds(row0, SUB)], sbuf.at[0])

        def body(i, carry):
            p = i & 1

            @pl.when(i > 1)
            def _():
                drain_scatters(ssem)

            pltpu.async_copy(dst_h.at[pl.ds(row0 + i * SUB, SUB)],
                             dbuf.at[p], dsem)

            @pl.when(i > 0)
            def _():
                drain_ibuf(sbuf.at[p], isem)

            g = [
                pltpu.async_copy(tab.at[sbuf.at[p, j]],
                                 rows_v.at[pl.ds(p * CHUNK + j * LANES, LANES)],
                                 gsem)
                for j in range(SUB)
            ]
            pltpu.async_copy(idx_h.at[pl.ds(row0 + (i + 1) * SUB, SUB)],
                             sbuf.at[1 - p], isem)
            for cp in g:
                cp.wait()
            drain_ibuf(dbuf.at[p], dsem)
            for j in range(SUB):
                pltpu.async_copy(rows_v.at[pl.ds(p * CHUNK + j * LANES, LANES)],
                                 acc.at[dbuf.at[p, j]], ssem, add=True)
            return carry

        lax.fori_loop(0, n_bodies, body, 0)
        drain_scatters(ssem)
        drain_scatters(ssem)
        drain_ibuf(sbuf.at[0], isem)
        plsc.subcore_barrier()
        pltpu.sync_copy(acc.at[pl.ds(zb, ZCH)], out_h.at[c, pl.ds(zb, ZCH)])

    return k(table, idx2d, dst2d, zeros)


BR = 1000


def _tc_layer1(P, W1p, b1):
    def body(p_ref, w_ref, b_ref, h_ref, dinv_ref):
        sblk = p_ref[0] + p_ref[1]
        dinv = 1.0 / jnp.maximum(sblk[:, 4:5], 1.0)
        mean = sblk * dinv
        h = jnp.dot(mean, w_ref[...], preferred_element_type=jnp.float32)
        h = jnp.maximum(h + b_ref[...], 0.0)
        h_ref[0] = h[:, :16]
        h_ref[1] = h[:, 16:]
        dinv_ref[...] = jnp.broadcast_to(dinv, (BR, 8))

    return pl.pallas_call(
        body,
        grid=(N // BR,),
        in_specs=[
            pl.BlockSpec((NC, BR, 16), lambda i: (0, i, 0)),
            pl.BlockSpec((16, HID), lambda i: (0, 0)),
            pl.BlockSpec((1, HID), lambda i: (0, 0)),
        ],
        out_specs=[
            pl.BlockSpec((NC, BR, 16), lambda i: (0, i, 0)),
            pl.BlockSpec((BR, 8), lambda i: (i, 0)),
        ],
        out_shape=[
            jax.ShapeDtypeStruct((NC, N, 16), jnp.float32),
            jax.ShapeDtypeStruct((N, 8), jnp.float32),
        ],
    )(P, W1p, b1)


def _tc_layer2(A, dinv8, W2s, b2):
    def body(a_ref, d_ref, w_ref, b_ref, o_ref):
        acc = jnp.dot(a_ref[0], w_ref[0], preferred_element_type=jnp.float32)
        acc += jnp.dot(a_ref[1], w_ref[1], preferred_element_type=jnp.float32)
        o_ref[...] = jnp.maximum(acc * d_ref[:, 0:1] + b_ref[...], 0.0)

    return pl.pallas_call(
        body,
        grid=(N // BR,),
        in_specs=[
            pl.BlockSpec((NC, BR, 16), lambda i: (0, i, 0)),
            pl.BlockSpec((BR, 8), lambda i: (i, 0)),
            pl.BlockSpec((NC, 16, HID), lambda i: (0, 0, 0)),
            pl.BlockSpec((1, HID), lambda i: (0, 0)),
        ],
        out_specs=pl.BlockSpec((BR, HID), lambda i: (i, 0)),
        out_shape=jax.ShapeDtypeStruct((N, HID), jnp.float32),
    )(A, dinv8, W2s, b2)


@jax.jit
def kernel(x, edge_index, W1, b1, W2, b2):
    src = edge_index[0]
    dst = edge_index[1]

    pad = EP + BODY - E
    src_p = jnp.concatenate([src, jnp.zeros((pad,), jnp.int32)])
    dst_p = jnp.concatenate([dst, jnp.full((pad,), N, jnp.int32)])
    src2d = src_p.reshape(IDXROWS + SUB, LANES)
    dst2d = dst_p.reshape(IDXROWS + SUB, LANES)

    zeros = jnp.zeros((NACC, 16), jnp.float32)

    xpad = jnp.pad(x, ((0, 0), (0, 16 - IN_DIM))).at[:, 4].set(1.0)

    P = _sc_agg(xpad.reshape(1, N, 16), src2d, dst2d, zeros, split_edges=True,
                rows_per_sub=IDXROWS // (NC * NS))

    W1p = jnp.pad(W1, ((0, 16 - IN_DIM), (0, 0)))
    h1t, dinv8 = _tc_layer1(P, W1p, b1.reshape(1, HID))

    A = _sc_agg(h1t, src2d, dst2d, zeros, split_edges=False,
                rows_per_sub=IDXROWS // NS)

    W2s = W2.reshape(NC, 16, HID)
    return _tc_layer2(A, dinv8, W2s, b2.reshape(1, HID))


# speedup vs baseline: 40.5933x; 1.0054x over previous
import functools
import jax
import jax.numpy as jnp
from jax import lax
from jax.experimental import pallas as pl
from jax.experimental.pallas import tpu as pltpu
from jax.experimental.pallas import tpu_sc as plsc

N = 100000
E = 6400000
IN_DIM = 4
HID = 32

NC = 2
NS = 16
LANES = 256
SUB = 3
CHUNK = SUB * LANES

BODY = CHUNK
EP = ((E + NC * NS * BODY - 1) // (NC * NS * BODY)) * (NC * NS * BODY)
IDXROWS = EP // LANES
NACC = ((N + 1) + NS * 8 - 1) // (NS * 8) * (NS * 8)
ZCH = NACC // NS

_MESH = plsc.VectorSubcoreMesh(core_axis_name="c", subcore_axis_name="s")


def _sc_agg(table, idx2d, dst2d, zeros, split_edges: bool, rows_per_sub: int):
    n_bodies = rows_per_sub // SUB
    stacked = table.shape[0] == NC

    @functools.partial(
        pl.kernel,
        out_type=jax.ShapeDtypeStruct((NC, NACC, 16), jnp.float32),
        mesh=_MESH,
        scratch_types=[
            pltpu.VMEM((2, SUB, LANES), jnp.int32),
            pltpu.VMEM((2, SUB, LANES), jnp.int32),
            pltpu.VMEM((2 * CHUNK, 16), jnp.float32),
            pltpu.VMEM_SHARED((NACC, 16), jnp.float32),
            pltpu.SemaphoreType.DMA,
            pltpu.SemaphoreType.DMA,
            pltpu.SemaphoreType.DMA,
            pltpu.SemaphoreType.DMA,
        ],
        compiler_params=pltpu.CompilerParams(use_tc_tiling_on_sc=False),
    )
    def k(table_h, idx_h, dst_h, zeros_h, out_h,
          sbuf, dbuf, rows_v, acc, gsem, ssem, isem, dsem):
        c = lax.axis_index("c")
        s = lax.axis_index("s")
        tab = table_h.at[c] if stacked else table_h.at[0]
        zb = s * ZCH
        pltpu.sync_copy(zeros_h.at[pl.ds(zb, ZCH)], acc.at[pl.ds(zb, ZCH)])
        plsc.subcore_barrier()

        if split_edges:
            row0 = (c * NS + s) * rows_per_sub
        else:
            row0 = s * rows_per_sub

        def drain_scatters(sem):
            for j in range(SUB):
                pltpu.make_async_copy(
                    zeros_h.at[pl.ds(0, LANES)],
                    rows_v.at[pl.ds(j * LANES, LANES)], sem).wait()

        def drain_ibuf(buf, sem):
            pltpu.make_async_copy(idx_h.at[pl.ds(row0, SUB)], buf, sem).wait()

        pltpu.sync_copy(idx_h.at[pl.ds(row0, SUB)], sbuf.at[0])

        def body(i, carry):
            p = i & 1

            @pl.when(i > 1)
            def _():
                drain_scatters(ssem)

            pltpu.async_copy(dst_h.at[pl.ds(row0 + i * SUB, SUB)],
                             dbuf.at[p], dsem)

            @pl.when(i > 0)
            def _():
                drain_ibuf(sbuf.at[p], isem)

            g = [
                pltpu.async_copy(tab.at[sbuf.at[p, j]],
                                 rows_v.at[pl.ds(p * CHUNK + j * LANES, LANES)],
                                 gsem)
                for j in range(SUB)
            ]
            pltpu.async_copy(idx_h.at[pl.ds(row0 + (i + 1) * SUB, SUB)],
                             sbuf.at[1 - p], isem)
            for cp in g:
                cp.wait()
            drain_ibuf(dbuf.at[p], dsem)
            for j in range(SUB):
                pltpu.async_copy(rows_v.at[pl.ds(p * CHUNK + j * LANES, LANES)],
                                 acc.at[dbuf.at[p, j]], ssem, add=True)
            return carry

        lax.fori_loop(0, n_bodies, body, 0)
        drain_scatters(ssem)
        drain_scatters(ssem)
        drain_ibuf(sbuf.at[0], isem)
        plsc.subcore_barrier()
        pltpu.sync_copy(acc.at[pl.ds(zb, ZCH)], out_h.at[c, pl.ds(zb, ZCH)])

    return k(table, idx2d, dst2d, zeros)


BR = 1000


def _tc_layer1(P, W1p, b1):
    def body(p_ref, w_ref, b_ref, h_ref, dinv_ref):
        sblk = p_ref[0] + p_ref[1]
        dinv = 1.0 / jnp.maximum(sblk[:, 4:5], 1.0)
        mean = sblk * dinv
        h = jnp.dot(mean, w_ref[...], preferred_element_type=jnp.float32)
        h = jnp.maximum(h + b_ref[...], 0.0)
        h_ref[0] = h[:, :16]
        h_ref[1] = h[:, 16:]
        dinv_ref[...] = jnp.broadcast_to(dinv, (BR, 8))

    return pl.pallas_call(
        body,
        grid=(N // BR,),
        in_specs=[
            pl.BlockSpec((NC, BR, 16), lambda i: (0, i, 0)),
            pl.BlockSpec((16, HID), lambda i: (0, 0)),
            pl.BlockSpec((1, HID), lambda i: (0, 0)),
        ],
        out_specs=[
            pl.BlockSpec((NC, BR, 16), lambda i: (0, i, 0)),
            pl.BlockSpec((BR, 8), lambda i: (i, 0)),
        ],
        out_shape=[
            jax.ShapeDtypeStruct((NC, N, 16), jnp.float32),
            jax.ShapeDtypeStruct((N, 8), jnp.float32),
        ],
    )(P, W1p, b1)


def _tc_layer2(A, dinv8, W2s, b2):
    def body(a_ref, d_ref, w_ref, b_ref, o_ref):
        acc = jnp.dot(a_ref[0], w_ref[0], preferred_element_type=jnp.float32)
        acc += jnp.dot(a_ref[1], w_ref[1], preferred_element_type=jnp.float32)
        o_ref[...] = jnp.maximum(acc * d_ref[:, 0:1] + b_ref[...], 0.0)

    return pl.pallas_call(
        body,
        grid=(N // BR,),
        in_specs=[
            pl.BlockSpec((NC, BR, 16), lambda i: (0, i, 0)),
            pl.BlockSpec((BR, 8), lambda i: (i, 0)),
            pl.BlockSpec((NC, 16, HID), lambda i: (0, 0, 0)),
            pl.BlockSpec((1, HID), lambda i: (0, 0)),
        ],
        out_specs=pl.BlockSpec((BR, HID), lambda i: (i, 0)),
        out_shape=jax.ShapeDtypeStruct((N, HID), jnp.float32),
    )(A, dinv8, W2s, b2)


@jax.jit
def kernel(x, edge_index, W1, b1, W2, b2):
    src = edge_index[0]
    dst = edge_index[1]

    pad = EP + BODY - E
    src_p = jnp.concatenate([src, jnp.zeros((pad,), jnp.int32)])
    dst_p = jnp.concatenate([dst, jnp.full((pad,), N, jnp.int32)])
    src2d = src_p.reshape(IDXROWS + SUB, LANES)
    dst2d = dst_p.reshape(IDXROWS + SUB, LANES)

    zeros = jnp.zeros((NACC, 16), jnp.float32)

    xpad = jnp.pad(x, ((0, 0), (0, 16 - IN_DIM))).at[:, 4].set(1.0)

    P = _sc_agg(xpad.reshape(1, N, 16), src2d, dst2d, zeros, split_edges=True,
                rows_per_sub=IDXROWS // (NC * NS))

    W1p = jnp.pad(W1, ((0, 16 - IN_DIM), (0, 0)))
    h1t, dinv8 = _tc_layer1(P, W1p, b1.reshape(1, HID))

    A = _sc_agg(h1t, src2d, dst2d, zeros, split_edges=False,
                rows_per_sub=IDXROWS // NS)

    W2s = W2.reshape(NC, 16, HID)
    return _tc_layer2(A, dinv8, W2s, b2.reshape(1, HID))
